# Initial kernel scaffold; baseline (speedup 1.0000x reference)
import jax
import jax.numpy as jnp
from jax.experimental import pallas as pl


def kernel(x, W):
    raise NotImplementedError("write your pallas kernel here")


# speedup vs baseline: 5.3720x; 5.3720x over previous
import jax
import jax.numpy as jnp
import numpy as np
from jax.experimental import pallas as pl

K = 256
S_TILE = 2048
MININT = np.int32(-(2 ** 31))


def _logits_kernel(x_ref, w_ref, out_ref):
    xt = x_ref[0]
    w = w_ref[...]
    out_ref[0] = jax.lax.dot_general(
        xt, w, (((1,), (1,)), ((), ())),
        preferred_element_type=jnp.float32)


def _finish_kernel(l_ref, mask_ref, probs_ref):
    l = l_ref[0]
    m = jnp.max(l, axis=0, keepdims=True)
    el = jnp.exp(l - m)
    ssum = jnp.sum(el, axis=0, keepdims=True)
    probs_ref[0] = el / ssum

    bits = jax.lax.bitcast_convert_type(l, jnp.int32)
    skey = jnp.where(bits < 0,
                     jnp.bitwise_xor(jnp.bitwise_not(bits), MININT),
                     bits)
    t = jnp.zeros((1, l.shape[1]), jnp.int32)
    for bit in range(31, -1, -1):
        step = MININT if bit == 31 else np.int32(1 << bit)
        cand = jnp.bitwise_or(t, step)
        cand_s = jnp.bitwise_xor(cand, MININT)
        cnt = jnp.sum((skey >= cand_s).astype(jnp.int32), axis=0,
                      keepdims=True)
        t = jnp.where(cnt >= K, cand, t)
    kth_s = jnp.bitwise_xor(t, MININT)
    mask_ref[0] = (skey >= kth_s).astype(jnp.float32)


def kernel(x, W):
    B, S, D = x.shape
    E = W.shape[0]
    logits = pl.pallas_call(
        _logits_kernel,
        grid=(B, S // S_TILE),
        in_specs=[pl.BlockSpec((1, S_TILE, D), lambda b, t: (b, t, 0)),
                  pl.BlockSpec((E, D), lambda b, t: (0, 0))],
        out_specs=pl.BlockSpec((1, S_TILE, E), lambda b, t: (b, t, 0)),
        out_shape=jax.ShapeDtypeStruct((B, S, E), jnp.float32),
    )(x, W)
    mask, probs = pl.pallas_call(
        _finish_kernel,
        grid=(B,),
        in_specs=[pl.BlockSpec((1, S, E), lambda b: (b, 0, 0))],
        out_specs=[pl.BlockSpec((1, S, E), lambda b: (b, 0, 0)),
                   pl.BlockSpec((1, S, E), lambda b: (b, 0, 0))],
        out_shape=[jax.ShapeDtypeStruct((B, S, E), jnp.float32),
                   jax.ShapeDtypeStruct((B, S, E), jnp.float32)],
    )(logits)
    return (mask, probs, logits)


# speedup vs baseline: 5.6772x; 1.0568x over previous
import jax
import jax.numpy as jnp
import numpy as np
from jax.experimental import pallas as pl
from jax.experimental.pallas import tpu as pltpu

K = 256
S_TILE = 2048
MININT = np.int32(-(2 ** 31))


def _logits_kernel(x_ref, w_ref, out_ref):
    xt = x_ref[0]
    w = w_ref[...]
    out_ref[0] = jax.lax.dot_general(
        xt, w, (((1,), (1,)), ((), ())),
        preferred_element_type=jnp.float32)


def _finish_kernel(l_ref, mask_ref, probs_ref):
    l2 = l_ref[...]
    NB, S2, E2 = l2.shape
    E = E2 // 2
    S = 2 * S2

    def both(v):
        return jnp.concatenate([v, v], axis=2)

    m128 = jnp.max(l2, axis=1, keepdims=True)
    m = jnp.maximum(m128[..., :E], m128[..., E:])
    el = jnp.exp(l2 - both(m))
    s128 = jnp.sum(el, axis=1, keepdims=True)
    ssum = s128[..., :E] + s128[..., E:]
    probs_ref[...] = el / both(ssum)

    bits = jax.lax.bitcast_convert_type(l2, jnp.int32)
    skey = jnp.where(bits < 0,
                     jnp.bitwise_xor(jnp.bitwise_not(bits), MININT),
                     bits)
    ukey1 = jax.lax.shift_right_logical(
        jnp.bitwise_xor(skey, MININT), 1)
    ukey1g = ukey1.reshape(NB, 8, S2 // 8, E2)

    t1 = jnp.zeros((NB, 1, E), jnp.int32)
    for bit in range(30, -1, -1):
        cand = jnp.bitwise_or(t1, np.int32(1 << bit))
        cand2 = both(cand)[:, :, None]
        neg = jax.lax.shift_right_arithmetic(ukey1g - cand2, 31)
        part = jnp.sum(neg, axis=2)
        cnt2 = jnp.sum(part, axis=1).reshape(NB, 1, E2)
        cnt_ge = np.int32(S) + cnt2[..., :E] + cnt2[..., E:]
        t1 = jnp.where(cnt_ge >= K, cand, t1)
    hi_s = jnp.bitwise_xor(jax.lax.shift_left(t1, 1), MININT)
    hi1_s = jnp.bitwise_or(hi_s, np.int32(1))
    cnt2 = jnp.sum((skey >= both(hi1_s)).astype(jnp.int32), axis=1,
                   keepdims=True)
    cnt_ge = cnt2[..., :E] + cnt2[..., E:]
    kth_s = jnp.where(cnt_ge >= K, hi1_s, hi_s)
    mask_ref[...] = (skey >= both(kth_s)).astype(jnp.float32)


def kernel(x, W):
    B, S, D = x.shape
    E = W.shape[0]
    logits = pl.pallas_call(
        _logits_kernel,
        grid=(B, S // S_TILE),
        in_specs=[pl.BlockSpec((1, S_TILE, D), lambda b, t: (b, t, 0)),
                  pl.BlockSpec((E, D), lambda b, t: (0, 0))],
        out_specs=pl.BlockSpec((1, S_TILE, E), lambda b, t: (b, t, 0)),
        out_shape=jax.ShapeDtypeStruct((B, S, E), jnp.float32),
        compiler_params=pltpu.CompilerParams(
            dimension_semantics=("parallel", "arbitrary")),
    )(x, W)
    l_packed = logits.reshape(B, S // 2, 2 * E)
    NB = 2
    mask2, probs2 = pl.pallas_call(
        _finish_kernel,
        grid=(B // NB,),
        in_specs=[pl.BlockSpec((NB, S // 2, 2 * E), lambda b: (b, 0, 0))],
        out_specs=[pl.BlockSpec((NB, S // 2, 2 * E), lambda b: (b, 0, 0)),
                   pl.BlockSpec((NB, S // 2, 2 * E), lambda b: (b, 0, 0))],
        out_shape=[jax.ShapeDtypeStruct((B, S // 2, 2 * E), jnp.float32),
                   jax.ShapeDtypeStruct((B, S // 2, 2 * E), jnp.float32)],
        compiler_params=pltpu.CompilerParams(
            dimension_semantics=("parallel",)),
    )(l_packed)
    return (mask2.reshape(B, S, E), probs2.reshape(B, S, E), logits)


# speedup vs baseline: 6.7417x; 1.1875x over previous
import jax
import jax.numpy as jnp
import numpy as np
from jax.experimental import pallas as pl
from jax.experimental.pallas import tpu as pltpu

K = 256
S_TILE = 2048
MININT = np.int32(-(2 ** 31))


def _logits_kernel(x_ref, w_ref, out_ref):
    xt = x_ref[0]
    w = w_ref[...]
    out_ref[0] = jax.lax.dot_general(
        xt, w, (((1,), (1,)), ((), ())),
        preferred_element_type=jnp.float32)


def _finish_kernel(l_ref, mask_ref, probs_ref):
    l = l_ref[0]
    S, E = l.shape
    S2 = S // 2
    m = jnp.max(l, axis=0, keepdims=True)
    el = jnp.exp(l - m)
    ssum = jnp.sum(el, axis=0, keepdims=True)
    probs_ref[0] = el / ssum

    bits = jax.lax.bitcast_convert_type(l, jnp.int32)
    skey = jnp.where(bits < 0,
                     jnp.bitwise_xor(jnp.bitwise_not(bits), MININT),
                     bits)
    skey2 = jnp.concatenate([skey[:S2], skey[S2:]], axis=1)
    ukey1 = jax.lax.shift_right_logical(
        jnp.bitwise_xor(skey2, MININT), 1)
    ukey1g = ukey1.reshape(8, S2 // 8, 2 * E)

    t1 = jnp.zeros((1, 2 * E), jnp.int32)
    for bit in range(30, -1, -1):
        cand = jnp.bitwise_or(t1, np.int32(1 << bit))
        neg = jax.lax.shift_right_arithmetic(ukey1g - cand[None], 31)
        part = jnp.sum(neg, axis=1)
        half = jnp.sum(part, axis=0, keepdims=True)
        cnt_ge = np.int32(S) + half + jnp.roll(half, E, axis=1)
        t1 = jnp.where(cnt_ge >= K, cand, t1)
    hi_s = jnp.bitwise_xor(jax.lax.shift_left(t1, 1), MININT)
    hi1_s = jnp.bitwise_or(hi_s, np.int32(1))
    cmp = (skey2 >= hi1_s).astype(jnp.int32)
    half = jnp.sum(cmp, axis=0, keepdims=True)
    cnt_ge = half + jnp.roll(half, E, axis=1)
    kth_s = jnp.where(cnt_ge >= K, hi1_s, hi_s)
    mask_ref[0] = (skey >= kth_s[:, :E]).astype(jnp.float32)


def kernel(x, W):
    B, S, D = x.shape
    E = W.shape[0]
    logits = pl.pallas_call(
        _logits_kernel,
        grid=(B, S // S_TILE),
        in_specs=[pl.BlockSpec((1, S_TILE, D), lambda b, t: (b, t, 0)),
                  pl.BlockSpec((E, D), lambda b, t: (0, 0))],
        out_specs=pl.BlockSpec((1, S_TILE, E), lambda b, t: (b, t, 0)),
        out_shape=jax.ShapeDtypeStruct((B, S, E), jnp.float32),
        compiler_params=pltpu.CompilerParams(
            dimension_semantics=("parallel", "arbitrary")),
    )(x, W)
    mask, probs = pl.pallas_call(
        _finish_kernel,
        grid=(B,),
        in_specs=[pl.BlockSpec((1, S, E), lambda b: (b, 0, 0))],
        out_specs=[pl.BlockSpec((1, S, E), lambda b: (b, 0, 0)),
                   pl.BlockSpec((1, S, E), lambda b: (b, 0, 0))],
        out_shape=[jax.ShapeDtypeStruct((B, S, E), jnp.float32),
                   jax.ShapeDtypeStruct((B, S, E), jnp.float32)],
        compiler_params=pltpu.CompilerParams(
            dimension_semantics=("parallel",)),
    )(logits)
    return (mask, probs, logits)


# speedup vs baseline: 7.0028x; 1.0387x over previous
import jax
import jax.numpy as jnp
import numpy as np
from jax.experimental import pallas as pl
from jax.experimental.pallas import tpu as pltpu

K = 256
S_TILE = 2048
MININT = np.int32(-(2 ** 31))


def _logits_kernel(x_ref, w_ref, out_ref, packed_ref):
    t = pl.program_id(1)
    xt = x_ref[0]
    w = w_ref[...]
    res = jax.lax.dot_general(
        xt, w, (((1,), (1,)), ((), ())),
        preferred_element_type=jnp.float32)
    out_ref[0] = res
    E = res.shape[1]

    @pl.when(t % 2 == 0)
    def _():
        packed_ref[0, :, :E] = res

    @pl.when(t % 2 == 1)
    def _():
        packed_ref[0, :, E:] = res


def _finish_kernel(l_ref, mask_ref, probs_ref):
    l2 = l_ref[...]
    NB, S2, E2 = l2.shape
    E = E2 // 2
    S = 2 * S2

    def halves(v):
        return v + jnp.roll(v, E, axis=2)

    m128 = jnp.max(l2, axis=1, keepdims=True)
    m = jnp.maximum(m128, jnp.roll(m128, E, axis=2))
    el2 = jnp.exp(l2 - m)
    ssum = halves(jnp.sum(el2, axis=1, keepdims=True))
    probs_ref[:, :S2, :] = el2[:, :, :E] / ssum[:, :, :E]
    probs_ref[:, S2:, :] = el2[:, :, E:] / ssum[:, :, E:]

    bits = jax.lax.bitcast_convert_type(l2, jnp.int32)
    skey2 = jnp.where(bits < 0,
                      jnp.bitwise_xor(jnp.bitwise_not(bits), MININT),
                      bits)
    ukey1 = jax.lax.shift_right_logical(
        jnp.bitwise_xor(skey2, MININT), 1)
    del bits, skey2
    ukey1g = ukey1.reshape(NB, 8, S2 // 8, E2)

    def key_to_float(ks):
        return jax.lax.bitcast_convert_type(
            jnp.where(ks >= 0, ks,
                      jnp.bitwise_not(jnp.bitwise_xor(ks, MININT))),
            jnp.float32)

    t1 = jnp.zeros((NB, 1, E2), jnp.int32)
    for bit in range(30, -1, -1):
        cand = jnp.bitwise_or(t1, np.int32(1 << bit))
        neg = jax.lax.shift_right_arithmetic(ukey1g - cand[:, None], 31)
        part = jnp.sum(neg, axis=2)
        half = jnp.sum(part, axis=1).reshape(NB, 1, E2)
        cnt_ge = np.int32(S) + halves(half)
        t1 = jnp.where(cnt_ge >= K, cand, t1)
    hi_s = jnp.bitwise_xor(jax.lax.shift_left(t1, 1), MININT)
    hi1_s = jnp.bitwise_or(hi_s, np.int32(1))
    cmp = (l2 >= key_to_float(hi1_s)).astype(jnp.int32)
    cnt_ge = halves(jnp.sum(cmp, axis=1, keepdims=True))
    kth_f = key_to_float(jnp.where(cnt_ge >= K, hi1_s, hi_s))
    mask_ref[:, :S2, :] = (l2[:, :, :E] >= kth_f[:, :, :E]
                           ).astype(jnp.float32)
    mask_ref[:, S2:, :] = (l2[:, :, E:] >= kth_f[:, :, E:]
                           ).astype(jnp.float32)


def kernel(x, W):
    B, S, D = x.shape
    E = W.shape[0]
    NT = S // S_TILE
    logits, l_packed = pl.pallas_call(
        _logits_kernel,
        grid=(B, NT),
        in_specs=[pl.BlockSpec(
                      (1, S_TILE, D),
                      lambda b, t: (b, (t % 2) * (NT // 2) + t // 2, 0)),
                  pl.BlockSpec((E, D), lambda b, t: (0, 0))],
        out_specs=[pl.BlockSpec(
                       (1, S_TILE, E),
                       lambda b, t: (b, (t % 2) * (NT // 2) + t // 2, 0)),
                   pl.BlockSpec((1, S_TILE, 2 * E),
                                lambda b, t: (b, t // 2, 0))],
        out_shape=[jax.ShapeDtypeStruct((B, S, E), jnp.float32),
                   jax.ShapeDtypeStruct((B, S // 2, 2 * E), jnp.float32)],
        compiler_params=pltpu.CompilerParams(
            dimension_semantics=("parallel", "arbitrary")),
    )(x, W)
    NB = 2
    mask, probs = pl.pallas_call(
        _finish_kernel,
        grid=(B // NB,),
        in_specs=[pl.BlockSpec((NB, S // 2, 2 * E), lambda b: (b, 0, 0))],
        out_specs=[pl.BlockSpec((NB, S, E), lambda b: (b, 0, 0)),
                   pl.BlockSpec((NB, S, E), lambda b: (b, 0, 0))],
        out_shape=[jax.ShapeDtypeStruct((B, S, E), jnp.float32),
                   jax.ShapeDtypeStruct((B, S, E), jnp.float32)],
        compiler_params=pltpu.CompilerParams(
            dimension_semantics=("parallel",)),
    )(l_packed)
    return (mask, probs, logits)


# speedup vs baseline: 7.9563x; 1.1361x over previous
import jax
import jax.numpy as jnp
import numpy as np
from jax.experimental import pallas as pl
from jax.experimental.pallas import tpu as pltpu

K = 256
S_TILE = 2048
MININT = np.int32(-(2 ** 31))


def _logits_kernel(x_ref, w_ref, out_ref, packed_ref):
    t = pl.program_id(1)
    xt = x_ref[0]
    w = w_ref[...]
    res = jax.lax.dot_general(
        xt, w, (((1,), (1,)), ((), ())),
        preferred_element_type=jnp.float32)
    out_ref[0] = res
    E = res.shape[1]

    @pl.when(t % 2 == 0)
    def _():
        packed_ref[0, :, :E] = res

    @pl.when(t % 2 == 1)
    def _():
        packed_ref[0, :, E:] = res


def _finish_kernel(l_ref, mask_ref, probs_ref):
    l2 = l_ref[...]
    NB, S2, E2 = l2.shape
    E = E2 // 2
    S = 2 * S2

    def halves(v):
        return v + jnp.roll(v, E, axis=2)

    m128 = jnp.max(l2, axis=1, keepdims=True)
    m = jnp.maximum(m128, jnp.roll(m128, E, axis=2))
    el2 = jnp.exp(l2 - m)
    ssum = halves(jnp.sum(el2, axis=1, keepdims=True))
    probs_ref[:, :S2, :] = el2[:, :, :E] / ssum[:, :, :E]
    probs_ref[:, S2:, :] = el2[:, :, E:] / ssum[:, :, E:]

    def key_to_float(ks):
        return jax.lax.bitcast_convert_type(
            jnp.where(ks >= 0, ks,
                      jnp.bitwise_not(jnp.bitwise_xor(ks, MININT))),
            jnp.float32)

    ones_row = jnp.ones((8, S2), jnp.bfloat16)
    t1 = jnp.zeros((NB, 1, E2), jnp.int32)
    kf = jnp.float32(K)
    for bit in range(31, -1, -1):
        step = MININT if bit == 31 else np.int32(1 << bit)
        cand = jnp.bitwise_or(t1, step)
        cand_f = key_to_float(jnp.bitwise_xor(cand, MININT))
        ind = (l2 >= cand_f).astype(jnp.bfloat16)
        cnts = [jax.lax.dot_general(
                    ones_row, ind[b], (((1,), (0,)), ((), ())),
                    preferred_element_type=jnp.float32)[:1]
                for b in range(NB)]
        cnt = jnp.stack(cnts, axis=0)
        cnt_ge = halves(cnt)
        t1 = jnp.where(cnt_ge >= kf, cand, t1)
    kth_f = key_to_float(jnp.bitwise_xor(t1, MININT))
    mask_ref[:, :S2, :] = (l2[:, :, :E] >= kth_f[:, :, :E]
                           ).astype(jnp.float32)
    mask_ref[:, S2:, :] = (l2[:, :, E:] >= kth_f[:, :, E:]
                           ).astype(jnp.float32)


def kernel(x, W):
    B, S, D = x.shape
    E = W.shape[0]
    NT = S // S_TILE
    logits, l_packed = pl.pallas_call(
        _logits_kernel,
        grid=(B, NT),
        in_specs=[pl.BlockSpec(
                      (1, S_TILE, D),
                      lambda b, t: (b, (t % 2) * (NT // 2) + t // 2, 0)),
                  pl.BlockSpec((E, D), lambda b, t: (0, 0))],
        out_specs=[pl.BlockSpec(
                       (1, S_TILE, E),
                       lambda b, t: (b, (t % 2) * (NT // 2) + t // 2, 0)),
                   pl.BlockSpec((1, S_TILE, 2 * E),
                                lambda b, t: (b, t // 2, 0))],
        out_shape=[jax.ShapeDtypeStruct((B, S, E), jnp.float32),
                   jax.ShapeDtypeStruct((B, S // 2, 2 * E), jnp.float32)],
        compiler_params=pltpu.CompilerParams(
            dimension_semantics=("parallel", "arbitrary")),
    )(x, W)
    NB = 2
    mask, probs = pl.pallas_call(
        _finish_kernel,
        grid=(B // NB,),
        in_specs=[pl.BlockSpec((NB, S // 2, 2 * E), lambda b: (b, 0, 0))],
        out_specs=[pl.BlockSpec((NB, S, E), lambda b: (b, 0, 0)),
                   pl.BlockSpec((NB, S, E), lambda b: (b, 0, 0))],
        out_shape=[jax.ShapeDtypeStruct((B, S, E), jnp.float32),
                   jax.ShapeDtypeStruct((B, S, E), jnp.float32)],
        compiler_params=pltpu.CompilerParams(
            dimension_semantics=("parallel",)),
    )(l_packed)
    return (mask, probs, logits)


# speedup vs baseline: 8.1489x; 1.0242x over previous
import jax
import jax.numpy as jnp
import numpy as np
from jax.experimental import pallas as pl
from jax.experimental.pallas import tpu as pltpu

K = 256
S_TILE = 4096
MININT = np.int32(-(2 ** 31))


def _logits_kernel(x_ref, w_ref, out_ref, packed_ref):
    t = pl.program_id(1)
    xt = x_ref[0]
    w = w_ref[...]
    res = jax.lax.dot_general(
        xt, w, (((1,), (1,)), ((), ())),
        preferred_element_type=jnp.float32)
    out_ref[0] = res
    E = res.shape[1]

    @pl.when(t % 2 == 0)
    def _():
        packed_ref[0, :, :E] = res

    @pl.when(t % 2 == 1)
    def _():
        packed_ref[0, :, E:] = res


def _finish_kernel(l_ref, mask_ref, probs_ref):
    l2 = l_ref[...]
    NB, S2, E2 = l2.shape
    E = E2 // 2
    S = 2 * S2

    def halves(v):
        return v + jnp.roll(v, E, axis=2)

    m128 = jnp.max(l2, axis=1, keepdims=True)
    m = jnp.maximum(m128, jnp.roll(m128, E, axis=2))
    el2 = jnp.exp(l2 - m)
    ssum = halves(jnp.sum(el2, axis=1, keepdims=True))
    probs_ref[:, :S2, :] = el2[:, :, :E] / ssum[:, :, :E]
    probs_ref[:, S2:, :] = el2[:, :, E:] / ssum[:, :, E:]

    def key_to_float(ks):
        return jax.lax.bitcast_convert_type(
            jnp.where(ks >= 0, ks,
                      jnp.bitwise_not(jnp.bitwise_xor(ks, MININT))),
            jnp.float32)

    ones_row = jnp.ones((8, S2), jnp.bfloat16)
    t1 = jnp.zeros((NB, 1, E2), jnp.int32)
    kf = jnp.float32(K)
    for bit in range(31, -1, -1):
        step = MININT if bit == 31 else np.int32(1 << bit)
        cand = jnp.bitwise_or(t1, step)
        cand_f = key_to_float(jnp.bitwise_xor(cand, MININT))
        ind = (l2 >= cand_f).astype(jnp.bfloat16)
        cnts = [jax.lax.dot_general(
                    ones_row, ind[b], (((1,), (0,)), ((), ())),
                    preferred_element_type=jnp.float32)[:1]
                for b in range(NB)]
        cnt = jnp.stack(cnts, axis=0)
        cnt_ge = halves(cnt)
        t1 = jnp.where(cnt_ge >= kf, cand, t1)
    kth_f = key_to_float(jnp.bitwise_xor(t1, MININT))
    mask_ref[:, :S2, :] = (l2[:, :, :E] >= kth_f[:, :, :E]
                           ).astype(jnp.float32)
    mask_ref[:, S2:, :] = (l2[:, :, E:] >= kth_f[:, :, E:]
                           ).astype(jnp.float32)


def kernel(x, W):
    B, S, D = x.shape
    E = W.shape[0]
    NT = S // S_TILE
    logits, l_packed = pl.pallas_call(
        _logits_kernel,
        grid=(B, NT),
        in_specs=[pl.BlockSpec(
                      (1, S_TILE, D),
                      lambda b, t: (b, (t % 2) * (NT // 2) + t // 2, 0)),
                  pl.BlockSpec((E, D), lambda b, t: (0, 0))],
        out_specs=[pl.BlockSpec(
                       (1, S_TILE, E),
                       lambda b, t: (b, (t % 2) * (NT // 2) + t // 2, 0)),
                   pl.BlockSpec((1, S_TILE, 2 * E),
                                lambda b, t: (b, t // 2, 0))],
        out_shape=[jax.ShapeDtypeStruct((B, S, E), jnp.float32),
                   jax.ShapeDtypeStruct((B, S // 2, 2 * E), jnp.float32)],
        compiler_params=pltpu.CompilerParams(
            dimension_semantics=("parallel", "arbitrary")),
    )(x, W)
    NB = 2
    mask, probs = pl.pallas_call(
        _finish_kernel,
        grid=(B // NB,),
        in_specs=[pl.BlockSpec((NB, S // 2, 2 * E), lambda b: (b, 0, 0))],
        out_specs=[pl.BlockSpec((NB, S, E), lambda b: (b, 0, 0)),
                   pl.BlockSpec((NB, S, E), lambda b: (b, 0, 0))],
        out_shape=[jax.ShapeDtypeStruct((B, S, E), jnp.float32),
                   jax.ShapeDtypeStruct((B, S, E), jnp.float32)],
        compiler_params=pltpu.CompilerParams(
            dimension_semantics=("parallel",)),
    )(l_packed)
    return (mask, probs, logits)


# speedup vs baseline: 8.2456x; 1.0119x over previous
import jax
import jax.numpy as jnp
import numpy as np
from jax.experimental import pallas as pl
from jax.experimental.pallas import tpu as pltpu

K = 256
S_TILE = 4096
MININT = np.int32(-(2 ** 31))


def _logits_kernel(x_ref, w_ref, out_ref):
    xt = x_ref[0]
    w = w_ref[...]
    out_ref[0] = jax.lax.dot_general(
        xt, w, (((1,), (1,)), ((), ())),
        preferred_element_type=jnp.float32)


def _finish_kernel(l_ref, mask_ref, probs_ref):
    l = l_ref[...]
    NB, S, E = l.shape
    S2 = S // 2
    E2 = 2 * E

    def halves(v):
        return v + jnp.roll(v, E, axis=2)

    l2 = jnp.concatenate([l[:, :S2], l[:, S2:]], axis=2)

    m128 = jnp.max(l2, axis=1, keepdims=True)
    m = jnp.maximum(m128, jnp.roll(m128, E, axis=2))
    ssum = halves(jnp.sum(jnp.exp(l2 - m), axis=1, keepdims=True))
    probs_ref[:, :S2, :] = jnp.exp(l2[:, :, :E] - m[:, :, :E]
                                   ) / ssum[:, :, :E]
    probs_ref[:, S2:, :] = jnp.exp(l2[:, :, E:] - m[:, :, E:]
                                   ) / ssum[:, :, E:]

    def key_to_float(ks):
        return jax.lax.bitcast_convert_type(
            jnp.where(ks >= 0, ks,
                      jnp.bitwise_not(jnp.bitwise_xor(ks, MININT))),
            jnp.float32)

    ones_row = jnp.ones((8, S2), jnp.bfloat16)
    t1 = jnp.zeros((NB, 1, E2), jnp.int32)
    kf = jnp.float32(K)
    for bit in range(31, -1, -1):
        step = MININT if bit == 31 else np.int32(1 << bit)
        cand = jnp.bitwise_or(t1, step)
        cand_f = key_to_float(jnp.bitwise_xor(cand, MININT))
        ind = (l2 >= cand_f).astype(jnp.bfloat16)
        cnts = [jax.lax.dot_general(
                    ones_row, ind[b], (((1,), (0,)), ((), ())),
                    preferred_element_type=jnp.float32)[:1]
                for b in range(NB)]
        cnt = jnp.stack(cnts, axis=0)
        cnt_ge = halves(cnt)
        t1 = jnp.where(cnt_ge >= kf, cand, t1)
    kth_f = key_to_float(jnp.bitwise_xor(t1, MININT))
    mask_ref[:, :S2, :] = (l2[:, :, :E] >= kth_f[:, :, :E]
                           ).astype(jnp.float32)
    mask_ref[:, S2:, :] = (l2[:, :, E:] >= kth_f[:, :, E:]
                           ).astype(jnp.float32)


def kernel(x, W):
    B, S, D = x.shape
    E = W.shape[0]
    logits = pl.pallas_call(
        _logits_kernel,
        grid=(B, S // S_TILE),
        in_specs=[pl.BlockSpec((1, S_TILE, D), lambda b, t: (b, t, 0)),
                  pl.BlockSpec((E, D), lambda b, t: (0, 0))],
        out_specs=pl.BlockSpec((1, S_TILE, E), lambda b, t: (b, t, 0)),
        out_shape=jax.ShapeDtypeStruct((B, S, E), jnp.float32),
        compiler_params=pltpu.CompilerParams(
            dimension_semantics=("parallel", "arbitrary")),
    )(x, W)
    NB = 2
    mask, probs = pl.pallas_call(
        _finish_kernel,
        grid=(B // NB,),
        in_specs=[pl.BlockSpec((NB, S, E), lambda b: (b, 0, 0))],
        out_specs=[pl.BlockSpec((NB, S, E), lambda b: (b, 0, 0)),
                   pl.BlockSpec((NB, S, E), lambda b: (b, 0, 0))],
        out_shape=[jax.ShapeDtypeStruct((B, S, E), jnp.float32),
                   jax.ShapeDtypeStruct((B, S, E), jnp.float32)],
        compiler_params=pltpu.CompilerParams(
            dimension_semantics=("parallel",)),
    )(logits)
    return (mask, probs, logits)
